# Initial kernel scaffold; baseline (speedup 1.0000x reference)
import jax
import jax.numpy as jnp
from jax.experimental import pallas as pl


def kernel(input_seq, weights):
    raise NotImplementedError("write your pallas kernel here")


# speedup vs baseline: 2.4511x; 2.4511x over previous
import jax
import jax.numpy as jnp
from jax.experimental import pallas as pl

_SEQ_BLOCK = 1024


def _bcast_kernel(w_ref, out_ref):
    w = w_ref[...]
    out_ref[...] = jnp.broadcast_to(w[None, :, :], out_ref.shape)


def kernel(input_seq, weights):
    batch, seq_len = input_seq.shape
    emb = weights.shape[1]
    grid = (seq_len // _SEQ_BLOCK,)
    out = pl.pallas_call(
        _bcast_kernel,
        grid=grid,
        in_specs=[pl.BlockSpec((_SEQ_BLOCK, emb), lambda j: (j, 0))],
        out_specs=pl.BlockSpec((batch, _SEQ_BLOCK, emb), lambda j: (0, j, 0)),
        out_shape=jax.ShapeDtypeStruct((batch, seq_len, emb), weights.dtype),
    )(weights)
    return out
